# baseline (speedup 1.0000x reference)
import functools

import jax
import jax.numpy as jnp
from jax import lax
from jax.experimental import pallas as pl
from jax.experimental.pallas import tpu as pltpu
from jax.experimental.pallas import tpu_sc as plsc

_B, _T, _D = 2, 2048, 2048
_E = 8
_K = 2
_F = 1024
_N = _B * _T
_NK = _N * _K
_TM = 512
_TMG = 256
_P = _NK + _E * _TMG
_G = _P // _TMG
_NW = 32
_CHUNK = _NK // _NW


def _router_body(x_ref, rw_ref, xbf_ref, eid_ref, val_ref, c1_ref, c2_ref):
    xb = x_ref[...]
    xbf_ref[...] = xb.astype(jnp.bfloat16)
    logits = jnp.dot(xb, rw_ref[...], preferred_element_type=jnp.float32)
    m = jnp.max(logits, axis=-1, keepdims=True)
    p = jnp.exp(logits - m)
    p = p / jnp.sum(p, axis=-1, keepdims=True)
    lane = lax.broadcasted_iota(jnp.int32, p.shape, 1)
    v1 = jnp.max(p, axis=-1, keepdims=True)
    i1 = jnp.min(jnp.where(p >= v1, lane, _E), axis=-1, keepdims=True)
    p2 = jnp.where(lane == i1, -1.0, p)
    v2 = jnp.max(p2, axis=-1, keepdims=True)
    i2 = jnp.min(jnp.where(p2 >= v2, lane, _E), axis=-1, keepdims=True)
    eid_ref[...] = jnp.concatenate([i1, i2], axis=1)
    val_ref[...] = jnp.concatenate([v1, v2], axis=1)
    oh1 = (i1 == lane).astype(jnp.int32)
    oh2 = (lane == i2).astype(jnp.int32)
    c1_ref[...] = jnp.concatenate(
        [jnp.sum(oh1[:_TM // 2], axis=0, keepdims=True),
         jnp.sum(oh1[_TM // 2:], axis=0, keepdims=True)], axis=0)[None]
    c2_ref[...] = jnp.concatenate(
        [jnp.sum(oh2[:_TM // 2], axis=0, keepdims=True),
         jnp.sum(oh2[_TM // 2:], axis=0, keepdims=True)], axis=0)[None]


def _router_call(x_flat, router_w):
    return pl.pallas_call(
        _router_body,
        grid=(_N // _TM,),
        in_specs=[
            pl.BlockSpec((_TM, _D), lambda i: (i, 0)),
            pl.BlockSpec((_D, _E), lambda i: (0, 0)),
        ],
        out_specs=[
            pl.BlockSpec((_TM, _D), lambda i: (i, 0)),
            pl.BlockSpec((_TM, _K), lambda i: (i, 0)),
            pl.BlockSpec((_TM, _K), lambda i: (i, 0)),
            pl.BlockSpec((1, 2, _E), lambda i: (i, 0, 0)),
            pl.BlockSpec((1, 2, _E), lambda i: (i, 0, 0)),
        ],
        out_shape=[
            jax.ShapeDtypeStruct((_N, _D), jnp.bfloat16),
            jax.ShapeDtypeStruct((_N, _K), jnp.int32),
            jax.ShapeDtypeStruct((_N, _K), jnp.float32),
            jax.ShapeDtypeStruct((_N // _TM, 2, _E), jnp.int32),
            jax.ShapeDtypeStruct((_N // _TM, 2, _E), jnp.int32),
        ],
        compiler_params=pltpu.CompilerParams(
            dimension_semantics=("arbitrary",)),
    )(x_flat, router_w)


def _shared_body(x_ref, sg_ref, su_ref, sd_ref, sh_ref):
    xb16 = x_ref[...].astype(jnp.bfloat16)
    h1 = jnp.dot(xb16, sg_ref[...], preferred_element_type=jnp.float32)
    h2 = jnp.dot(xb16, su_ref[...], preferred_element_type=jnp.float32)
    h = h1 * (1.0 / (1.0 + jnp.exp(-h1))) * h2
    sh_ref[...] = jnp.dot(h.astype(jnp.bfloat16), sd_ref[...],
                          preferred_element_type=jnp.float32)


def _shared_call(x_flat, sg16, su16, sd16):
    return pl.pallas_call(
        _shared_body,
        grid=(_N // _TM,),
        in_specs=[
            pl.BlockSpec((_TM, _D), lambda i: (i, 0)),
            pl.BlockSpec((_D, _F), lambda i: (0, 0)),
            pl.BlockSpec((_D, _F), lambda i: (0, 0)),
            pl.BlockSpec((_F, _D), lambda i: (0, 0)),
        ],
        out_specs=pl.BlockSpec((_TM, _D), lambda i: (i, 0)),
        out_shape=jax.ShapeDtypeStruct((_N, _D), jnp.float32),
        compiler_params=pltpu.CompilerParams(
            dimension_semantics=("arbitrary",)),
    )(x_flat, sg16, su16, sd16)


def _dispatch_body(eid_hbm, val_hbm, cnt_hbm, inv_hbm, tok_hbm, w_hbm, tex_hbm,
                   eidv, valv, cntv, wbbuf, offsbuf, histbuf, padbuf, evbuf,
                   slot_a, slot_b, tok_a, tok_b, val_a, val_b,
                   padidx, zb, texv, sem):
    cid = lax.axis_index("c")
    sid = lax.axis_index("s")
    wid = sid * 2 + cid
    base = wid * _CHUNK
    lane = lax.broadcasted_iota(jnp.int32, (16,), 0)
    zero = jnp.zeros((16,), jnp.int32)
    one = zero + 1
    widv = zero + wid

    pltpu.sync_copy(eid_hbm.at[pl.ds(base, _CHUNK)], eidv)
    pltpu.sync_copy(val_hbm.at[pl.ds(base, _CHUNK)], valv)
    pltpu.sync_copy(cnt_hbm, cntv)
    zb[...] = zero

    lane8 = lane & 7
    hist = zero
    pre = zero
    for i in range(_NW // 2):
        vlo = plsc.load_gather(cntv, [lane8 + i * 16])
        vhi = plsc.load_gather(cntv, [lane8 + i * 16 + 8])
        hist = hist + vlo + vhi
        pre = pre + jnp.where(widv > 2 * i, vlo, zero)
        pre = pre + jnp.where(widv > 2 * i + 1, vhi, zero)

    padded = ((hist + (_TMG - 1)) >> 8) << 8
    padded = jnp.where(lane < _E, padded, zero)
    csum = plsc.cumsum(padded)
    offs = csum - padded
    wbbuf[...] = offs + pre
    offsbuf[...] = offs
    histbuf[...] = hist
    padbuf[...] = padded
    tots = plsc.load_gather(offsbuf, [zero + _E])

    for v in range(_CHUNK // 16):
        ev = eidv[pl.ds(v * 16, 16)]
        evbuf[pl.ds(0, 16)] = zero - 1
        evbuf[pl.ds(16, 16)] = ev
        rank = zero
        for s in range(1, 16):
            sh = plsc.load_gather(evbuf, [lane + (16 - s)])
            rank = rank + jnp.where(ev == sh, 1, 0)
        slot = plsc.load_gather(wbbuf, [ev]) + rank
        plsc.addupdate_scatter(wbbuf, [ev], one)
        half = slot_a if v < 8 else slot_b
        tokh = tok_a if v < 8 else tok_b
        valh = val_a if v < 8 else val_b
        o = (v % 8) * 16
        half[pl.ds(o, 16)] = slot
        tokh[pl.ds(o, 16)] = (base + v * 16 + lane) & (_N - 1)
        valh[pl.ds(o, 16)] = valv[pl.ds(v * 16, 16)]

    pltpu.sync_copy(slot_a, inv_hbm.at[pl.ds(base, 128)])
    pltpu.sync_copy(slot_b, inv_hbm.at[pl.ds(base + 128, 128)])
    d1 = pltpu.async_copy(tok_a, tok_hbm.at[slot_a], sem)
    d2 = pltpu.async_copy(tok_b, tok_hbm.at[slot_b], sem)
    d3 = pltpu.async_copy(val_a, w_hbm.at[slot_a], sem)
    d4 = pltpu.async_copy(val_b, w_hbm.at[slot_b], sem)
    d1.wait()
    d2.wait()
    d3.wait()
    d4.wait()

    thr = [plsc.load_gather(offsbuf, [zero + e]) >> 8 for e in range(1, _E + 1)]
    for v in range(3):
        g = lane + v * 16
        ex = zero
        for t in thr:
            ex = ex + jnp.where(g >= t, 1, 0)
        texv[pl.ds(v * 16, 16)] = jnp.minimum(ex, _E - 1)
    pltpu.sync_copy(texv, tex_hbm)


def _dispatch_call(eid_km, val_km, counts):
    mesh = plsc.VectorSubcoreMesh(core_axis_name="c", subcore_axis_name="s", num_cores=2, num_subcores=16)
    f = pl.kernel(
        _dispatch_body,
        out_type=(
            jax.ShapeDtypeStruct((_NK,), jnp.int32),
            jax.ShapeDtypeStruct((_P,), jnp.int32),
            jax.ShapeDtypeStruct((_P,), jnp.float32),
            jax.ShapeDtypeStruct((48,), jnp.int32),
        ),
        mesh=mesh,
        compiler_params=pltpu.CompilerParams(needs_layout_passes=False),
        scratch_types=[
            pltpu.VMEM((_CHUNK,), jnp.int32),
            pltpu.VMEM((_CHUNK,), jnp.float32),
            pltpu.VMEM((_NW * _E,), jnp.int32),
            pltpu.VMEM((16,), jnp.int32),
            pltpu.VMEM((16,), jnp.int32),
            pltpu.VMEM((16,), jnp.int32),
            pltpu.VMEM((16,), jnp.int32),
            pltpu.VMEM((32,), jnp.int32),
            pltpu.VMEM((128,), jnp.int32),
            pltpu.VMEM((128,), jnp.int32),
            pltpu.VMEM((128,), jnp.int32),
            pltpu.VMEM((128,), jnp.int32),
            pltpu.VMEM((128,), jnp.float32),
            pltpu.VMEM((128,), jnp.float32),
            pltpu.VMEM((16,), jnp.int32),
            pltpu.VMEM((16,), jnp.int32),
            pltpu.VMEM((48,), jnp.int32),
            pltpu.SemaphoreType.DMA,
        ],
    )
    return f(eid_km, val_km, counts)


_GR = 32


def _gather_body(x_hbm, tok_hbm, xs_hbm,
                 idx0, idx1, rows0, rows1, gs0, gs1, ws0, ws1):
    cid = lax.axis_index("c")
    sid = lax.axis_index("s")
    wid = sid * 2 + cid
    rbase = wid * (_P // _NW)
    nch = _P // _NW // _GR
    idxb = [idx0, idx1]
    rowb = [rows0, rows1]
    gsem = [gs0, gs1]
    wsem = [ws0, ws1]
    gdesc = [None, None]
    wdesc = [None, None]

    lane = lax.broadcasted_iota(jnp.int32, (16,), 0)

    def load_idx(b, buf):
        pltpu.sync_copy(tok_hbm.at[pl.ds(b, _GR)], buf)
        for u in range(_GR // 16):
            t = buf[pl.ds(u * 16, 16)]
            buf[pl.ds(u * 16, 16)] = jnp.minimum(jnp.maximum(t, 0), _N - 1)

    load_idx(rbase, idxb[0])
    gdesc[0] = pltpu.async_copy(x_hbm.at[idxb[0]], rowb[0], gsem[0])
    for c in range(nch):
        cb = c % 2
        nb = (c + 1) % 2
        if c + 1 < nch:
            if wdesc[nb] is not None:
                wdesc[nb].wait()
            load_idx(rbase + (c + 1) * _GR, idxb[nb])
            gdesc[nb] = pltpu.async_copy(x_hbm.at[idxb[nb]], rowb[nb],
                                         gsem[nb])
        gdesc[cb].wait()
        wdesc[cb] = pltpu.async_copy(rowb[cb],
                                     xs_hbm.at[pl.ds(rbase + c * _GR, _GR)],
                                     wsem[cb])
    for d in wdesc:
        if d is not None:
            d.wait()


def _gather_call(xbf_i32, tok_slot):
    mesh = plsc.VectorSubcoreMesh(core_axis_name="c", subcore_axis_name="s", num_cores=2, num_subcores=16)
    f = pl.kernel(
        _gather_body,
        out_type=jax.ShapeDtypeStruct((_P, _D // 2), jnp.int32),
        mesh=mesh,
        compiler_params=pltpu.CompilerParams(needs_layout_passes=False),
        scratch_types=[
            pltpu.VMEM((_GR,), jnp.int32),
            pltpu.VMEM((_GR,), jnp.int32),
            pltpu.VMEM((_GR, _D // 2), jnp.int32),
            pltpu.VMEM((_GR, _D // 2), jnp.int32),
            pltpu.SemaphoreType.DMA,
            pltpu.SemaphoreType.DMA,
            pltpu.SemaphoreType.DMA,
            pltpu.SemaphoreType.DMA,
        ],
    )
    return f(xbf_i32, tok_slot)


def _group_body(ex_ref, xs_ref, g_ref, u_ref, d_ref, w_ref, y_ref):
    xb = xs_ref[...].astype(jnp.bfloat16)
    h1 = jnp.dot(xb, g_ref[0], preferred_element_type=jnp.float32)
    h2 = jnp.dot(xb, u_ref[0], preferred_element_type=jnp.float32)
    h = h1 * (1.0 / (1.0 + jnp.exp(-h1))) * h2 * w_ref[...]
    y_ref[...] = jnp.dot(h.astype(jnp.bfloat16), d_ref[0],
                         preferred_element_type=jnp.float32)


def _group_call(tile_ex, xs, gw16, uw16, dw16, w_slot):
    grid_spec = pltpu.PrefetchScalarGridSpec(
        num_scalar_prefetch=1,
        grid=(_G,),
        in_specs=[
            pl.BlockSpec((_TMG, _D), lambda i, ex: (i, 0)),
            pl.BlockSpec((1, _D, _F), lambda i, ex: (ex[i], 0, 0)),
            pl.BlockSpec((1, _D, _F), lambda i, ex: (ex[i], 0, 0)),
            pl.BlockSpec((1, _F, _D), lambda i, ex: (ex[i], 0, 0)),
            pl.BlockSpec((_TMG, 1), lambda i, ex: (i, 0)),
        ],
        out_specs=pl.BlockSpec((_TMG, _D), lambda i, ex: (i, 0)),
    )
    return pl.pallas_call(
        _group_body,
        grid_spec=grid_spec,
        out_shape=jax.ShapeDtypeStruct((_P, _D), jnp.float32),
        compiler_params=pltpu.CompilerParams(
            dimension_semantics=("arbitrary",)),
    )(tile_ex, xs, gw16, uw16, dw16, w_slot)


_CT = 8


def _combine_body(sh_hbm, y_hbm, inv_hbm, out_hbm,
                  idxa, idxb_, sh0, sh1, y0, y1, gs0, gs1, ws0, ws1):
    cid = lax.axis_index("c")
    sid = lax.axis_index("s")
    wid = sid * 2 + cid
    tbase = wid * (_N // _NW)
    nch = _N // _NW // _CT
    idxv = [idxa, idxb_]
    shb = [sh0, sh1]
    yb = [y0, y1]
    gsem = [gs0, gs1]
    wsem = [ws0, ws1]
    gdesc = [None, None]
    wdesc = [None, None]

    def stage(c, b):
        tb = tbase + c * _CT
        pltpu.sync_copy(inv_hbm.at[pl.ds(tb, _CT)], idxv[b].at[pl.ds(0, _CT)])
        pltpu.sync_copy(inv_hbm.at[pl.ds(_N + tb, _CT)],
                        idxv[b].at[pl.ds(_CT, _CT)])
        pltpu.sync_copy(sh_hbm.at[pl.ds(tb, _CT)], shb[b])
        return pltpu.async_copy(y_hbm.at[idxv[b]], yb[b], gsem[b])

    gdesc[0] = stage(0, 0)
    for c in range(nch):
        cb = c % 2
        nb = (c + 1) % 2
        if c + 1 < nch:
            if wdesc[nb] is not None:
                wdesc[nb].wait()
            gdesc[nb] = stage(c + 1, nb)
        gdesc[cb].wait()
        shv = shb[cb]
        yv = yb[cb]

        def add_rows(i, _, shv=shv, yv=yv):
            t = i // (_D // 16)
            cv = i % (_D // 16)
            s = (shv[t, pl.ds(cv * 16, 16)]
                 + yv[t, pl.ds(cv * 16, 16)]
                 + yv[t + _CT, pl.ds(cv * 16, 16)])
            shv[t, pl.ds(cv * 16, 16)] = s
            return 0

        lax.fori_loop(0, _CT * (_D // 16), add_rows, 0)
        wdesc[cb] = pltpu.async_copy(
            shv, out_hbm.at[pl.ds(tbase + c * _CT, _CT)], wsem[cb])
    for d in wdesc:
        if d is not None:
            d.wait()


def _combine_call(sh, y, inv):
    mesh = plsc.VectorSubcoreMesh(core_axis_name="c", subcore_axis_name="s", num_cores=2, num_subcores=16)
    f = pl.kernel(
        _combine_body,
        out_type=jax.ShapeDtypeStruct((_N, _D), jnp.float32),
        mesh=mesh,
        compiler_params=pltpu.CompilerParams(needs_layout_passes=False),
        scratch_types=[
            pltpu.VMEM((2 * _CT,), jnp.int32),
            pltpu.VMEM((2 * _CT,), jnp.int32),
            pltpu.VMEM((_CT, _D), jnp.float32),
            pltpu.VMEM((_CT, _D), jnp.float32),
            pltpu.VMEM((2 * _CT, _D), jnp.float32),
            pltpu.VMEM((2 * _CT, _D), jnp.float32),
            pltpu.SemaphoreType.DMA,
            pltpu.SemaphoreType.DMA,
            pltpu.SemaphoreType.DMA,
            pltpu.SemaphoreType.DMA,
        ],
    )
    return f(sh, y, inv)


@jax.jit
def kernel(x, router_w, shared_gate, shared_up, shared_down, gate_w, up_w, down_w):
    x_flat = x.reshape(_N, _D)
    sg16 = shared_gate.astype(jnp.bfloat16)
    su16 = shared_up.astype(jnp.bfloat16)
    sd16 = shared_down.astype(jnp.bfloat16)
    gw16 = gate_w.astype(jnp.bfloat16)
    uw16 = up_w.astype(jnp.bfloat16)
    dw16 = down_w.astype(jnp.bfloat16)

    xbf, eid2, val2, c1, c2 = _router_call(x_flat, router_w)
    eid_km = eid2.T.reshape(_NK)
    val_km = val2.T.reshape(_NK)
    counts = jnp.concatenate([c1.reshape(_N // 256, _E), c2.reshape(_N // 256, _E)], axis=0).reshape(_NW * _E)
    inv, tok_slot, w_slot, tile_ex = _dispatch_call(eid_km, val_km, counts)
    xbf_i32 = lax.bitcast_convert_type(xbf.reshape(_N, _D // 2, 2), jnp.int32)
    xs_i32 = _gather_call(xbf_i32, tok_slot)
    xs = lax.bitcast_convert_type(xs_i32, jnp.bfloat16).reshape(_P, _D)
    y = _group_call(tile_ex, xs, gw16, uw16, dw16, w_slot.reshape(_P, 1))
    sh = _shared_call(x_flat, sg16, su16, sd16)
    out = _combine_call(sh, y, inv)
    return out.reshape(_B, _T, _D)


# speedup vs baseline: 1.6633x; 1.6633x over previous
import functools

import jax
import jax.numpy as jnp
from jax import lax
from jax.experimental import pallas as pl
from jax.experimental.pallas import tpu as pltpu
from jax.experimental.pallas import tpu_sc as plsc

_B, _T, _D = 2, 2048, 2048
_E = 8
_K = 2
_F = 1024
_N = _B * _T
_NK = _N * _K
_TM = 512
_TMG = 256
_P = _NK + _E * _TMG
_G = _P // _TMG
_NW = 32
_CHUNK = _NK // _NW


def _router_body(x_ref, rw_ref, eid_ref, val_ref, c1_ref, c2_ref):
    xb = x_ref[...]
    logits = jnp.dot(xb, rw_ref[...], preferred_element_type=jnp.float32)
    m = jnp.max(logits, axis=-1, keepdims=True)
    p = jnp.exp(logits - m)
    p = p / jnp.sum(p, axis=-1, keepdims=True)
    lane = lax.broadcasted_iota(jnp.int32, p.shape, 1)
    v1 = jnp.max(p, axis=-1, keepdims=True)
    i1 = jnp.min(jnp.where(p >= v1, lane, _E), axis=-1, keepdims=True)
    p2 = jnp.where(lane == i1, -1.0, p)
    v2 = jnp.max(p2, axis=-1, keepdims=True)
    i2 = jnp.min(jnp.where(p2 >= v2, lane, _E), axis=-1, keepdims=True)
    eid_ref[...] = jnp.concatenate([i1, i2], axis=1)
    val_ref[...] = jnp.concatenate([v1, v2], axis=1)
    oh1 = (i1 == lane).astype(jnp.int32)
    oh2 = (lane == i2).astype(jnp.int32)
    c1_ref[...] = jnp.concatenate(
        [jnp.sum(oh1[:_TM // 2], axis=0, keepdims=True),
         jnp.sum(oh1[_TM // 2:], axis=0, keepdims=True)], axis=0)[None]
    c2_ref[...] = jnp.concatenate(
        [jnp.sum(oh2[:_TM // 2], axis=0, keepdims=True),
         jnp.sum(oh2[_TM // 2:], axis=0, keepdims=True)], axis=0)[None]


def _router_call(x_flat, router_w):
    return pl.pallas_call(
        _router_body,
        grid=(_N // _TM,),
        in_specs=[
            pl.BlockSpec((_TM, _D), lambda i: (i, 0)),
            pl.BlockSpec((_D, _E), lambda i: (0, 0)),
        ],
        out_specs=[
            pl.BlockSpec((_TM, _K), lambda i: (i, 0)),
            pl.BlockSpec((_TM, _K), lambda i: (i, 0)),
            pl.BlockSpec((1, 2, _E), lambda i: (i, 0, 0)),
            pl.BlockSpec((1, 2, _E), lambda i: (i, 0, 0)),
        ],
        out_shape=[
            jax.ShapeDtypeStruct((_N, _K), jnp.int32),
            jax.ShapeDtypeStruct((_N, _K), jnp.float32),
            jax.ShapeDtypeStruct((_N // _TM, 2, _E), jnp.int32),
            jax.ShapeDtypeStruct((_N // _TM, 2, _E), jnp.int32),
        ],
        compiler_params=pltpu.CompilerParams(
            dimension_semantics=("arbitrary",)),
    )(x_flat, router_w)


def _shared_body(x_ref, sg_ref, su_ref, sd_ref, sh_ref):
    xb16 = x_ref[...].astype(jnp.bfloat16)
    h1 = jnp.dot(xb16, sg_ref[...], preferred_element_type=jnp.float32)
    h2 = jnp.dot(xb16, su_ref[...], preferred_element_type=jnp.float32)
    h = h1 * (1.0 / (1.0 + jnp.exp(-h1))) * h2
    sh_ref[...] = jnp.dot(h.astype(jnp.bfloat16), sd_ref[...],
                          preferred_element_type=jnp.float32)


def _shared_call(x_flat, sg16, su16, sd16):
    return pl.pallas_call(
        _shared_body,
        grid=(_N // _TM,),
        in_specs=[
            pl.BlockSpec((_TM, _D), lambda i: (i, 0)),
            pl.BlockSpec((_D, _F), lambda i: (0, 0)),
            pl.BlockSpec((_D, _F), lambda i: (0, 0)),
            pl.BlockSpec((_F, _D), lambda i: (0, 0)),
        ],
        out_specs=pl.BlockSpec((_TM, _D), lambda i: (i, 0)),
        out_shape=jax.ShapeDtypeStruct((_N, _D), jnp.float32),
        compiler_params=pltpu.CompilerParams(
            dimension_semantics=("arbitrary",)),
    )(x_flat, sg16, su16, sd16)


def _dispatch_body(eid_hbm, val_hbm, cnt_hbm, inv_hbm, tok_hbm, w_hbm, tex_hbm,
                   eidv, valv, cntv, wbbuf, offsbuf, histbuf, padbuf, evbuf,
                   slot_a, slot_b, tok_a, tok_b, val_a, val_b,
                   padidx, zb, texv, sem):
    cid = lax.axis_index("c")
    sid = lax.axis_index("s")
    wid = sid * 2 + cid
    base = wid * _CHUNK
    lane = lax.broadcasted_iota(jnp.int32, (16,), 0)
    zero = jnp.zeros((16,), jnp.int32)
    one = zero + 1
    widv = zero + wid

    pltpu.sync_copy(eid_hbm.at[pl.ds(base, _CHUNK)], eidv)
    pltpu.sync_copy(val_hbm.at[pl.ds(base, _CHUNK)], valv)
    pltpu.sync_copy(cnt_hbm, cntv)
    zb[...] = zero

    lane8 = lane & 7
    hist = zero
    pre = zero
    for i in range(_NW // 2):
        vlo = plsc.load_gather(cntv, [lane8 + i * 16])
        vhi = plsc.load_gather(cntv, [lane8 + i * 16 + 8])
        hist = hist + vlo + vhi
        pre = pre + jnp.where(widv > 2 * i, vlo, zero)
        pre = pre + jnp.where(widv > 2 * i + 1, vhi, zero)

    padded = ((hist + (_TMG - 1)) >> 8) << 8
    padded = jnp.where(lane < _E, padded, zero)
    csum = plsc.cumsum(padded)
    offs = csum - padded
    wbbuf[...] = offs + pre
    offsbuf[...] = offs
    histbuf[...] = hist
    padbuf[...] = padded
    tots = plsc.load_gather(offsbuf, [zero + _E])

    for v in range(_CHUNK // 16):
        ev = eidv[pl.ds(v * 16, 16)]
        evbuf[pl.ds(0, 16)] = zero - 1
        evbuf[pl.ds(16, 16)] = ev
        rank = zero
        for s in range(1, 16):
            sh = plsc.load_gather(evbuf, [lane + (16 - s)])
            rank = rank + jnp.where(ev == sh, 1, 0)
        slot = plsc.load_gather(wbbuf, [ev]) + rank
        plsc.addupdate_scatter(wbbuf, [ev], one)
        half = slot_a if v < 8 else slot_b
        tokh = tok_a if v < 8 else tok_b
        valh = val_a if v < 8 else val_b
        o = (v % 8) * 16
        half[pl.ds(o, 16)] = slot
        tokh[pl.ds(o, 16)] = (base + v * 16 + lane) & (_N - 1)
        valh[pl.ds(o, 16)] = valv[pl.ds(v * 16, 16)]

    pltpu.sync_copy(slot_a, inv_hbm.at[pl.ds(base, 128)])
    pltpu.sync_copy(slot_b, inv_hbm.at[pl.ds(base + 128, 128)])
    d1 = pltpu.async_copy(tok_a, tok_hbm.at[slot_a], sem)
    d2 = pltpu.async_copy(tok_b, tok_hbm.at[slot_b], sem)
    d3 = pltpu.async_copy(val_a, w_hbm.at[slot_a], sem)
    d4 = pltpu.async_copy(val_b, w_hbm.at[slot_b], sem)
    d1.wait()
    d2.wait()
    d3.wait()
    d4.wait()

    thr = [plsc.load_gather(offsbuf, [zero + e]) >> 8 for e in range(1, _E + 1)]
    for v in range(3):
        g = lane + v * 16
        ex = zero
        for t in thr:
            ex = ex + jnp.where(g >= t, 1, 0)
        texv[pl.ds(v * 16, 16)] = jnp.minimum(ex, _E - 1)
    pltpu.sync_copy(texv, tex_hbm)


def _dispatch_call(eid_km, val_km, counts):
    mesh = plsc.VectorSubcoreMesh(core_axis_name="c", subcore_axis_name="s", num_cores=2, num_subcores=16)
    f = pl.kernel(
        _dispatch_body,
        out_type=(
            jax.ShapeDtypeStruct((_NK,), jnp.int32),
            jax.ShapeDtypeStruct((_P,), jnp.int32),
            jax.ShapeDtypeStruct((_P,), jnp.float32),
            jax.ShapeDtypeStruct((48,), jnp.int32),
        ),
        mesh=mesh,
        compiler_params=pltpu.CompilerParams(needs_layout_passes=False),
        scratch_types=[
            pltpu.VMEM((_CHUNK,), jnp.int32),
            pltpu.VMEM((_CHUNK,), jnp.float32),
            pltpu.VMEM((_NW * _E,), jnp.int32),
            pltpu.VMEM((16,), jnp.int32),
            pltpu.VMEM((16,), jnp.int32),
            pltpu.VMEM((16,), jnp.int32),
            pltpu.VMEM((16,), jnp.int32),
            pltpu.VMEM((32,), jnp.int32),
            pltpu.VMEM((128,), jnp.int32),
            pltpu.VMEM((128,), jnp.int32),
            pltpu.VMEM((128,), jnp.int32),
            pltpu.VMEM((128,), jnp.int32),
            pltpu.VMEM((128,), jnp.float32),
            pltpu.VMEM((128,), jnp.float32),
            pltpu.VMEM((16,), jnp.int32),
            pltpu.VMEM((16,), jnp.int32),
            pltpu.VMEM((48,), jnp.int32),
            pltpu.SemaphoreType.DMA,
        ],
    )
    return f(eid_km, val_km, counts)


_GR = 16


def _gather_body(x_hbm, tok_hbm, xs_hbm,
                 idx0, idx1, rows0, rows1, gs0, gs1, ws0, ws1):
    cid = lax.axis_index("c")
    sid = lax.axis_index("s")
    wid = sid * 2 + cid
    rbase = wid * (_P // _NW)
    nch = _P // _NW // _GR
    idxb = [idx0, idx1]
    rowb = [rows0, rows1]
    gsem = [gs0, gs1]
    wsem = [ws0, ws1]
    gdesc = [None, None]
    wdesc = [None, None]

    lane = lax.broadcasted_iota(jnp.int32, (16,), 0)

    def load_idx(b, buf):
        pltpu.sync_copy(tok_hbm.at[pl.ds(b, _GR)], buf)
        for u in range(_GR // 16):
            t = buf[pl.ds(u * 16, 16)]
            buf[pl.ds(u * 16, 16)] = jnp.minimum(jnp.maximum(t, 0), _N - 1)

    load_idx(rbase, idxb[0])
    gdesc[0] = pltpu.async_copy(x_hbm.at[idxb[0]], rowb[0], gsem[0])
    for c in range(nch):
        cb = c % 2
        nb = (c + 1) % 2
        if c + 1 < nch:
            if wdesc[nb] is not None:
                wdesc[nb].wait()
            load_idx(rbase + (c + 1) * _GR, idxb[nb])
            gdesc[nb] = pltpu.async_copy(x_hbm.at[idxb[nb]], rowb[nb],
                                         gsem[nb])
        gdesc[cb].wait()
        wdesc[cb] = pltpu.async_copy(rowb[cb],
                                     xs_hbm.at[pl.ds(rbase + c * _GR, _GR)],
                                     wsem[cb])
    for d in wdesc:
        if d is not None:
            d.wait()


def _gather_call(x_flat, tok_slot):
    mesh = plsc.VectorSubcoreMesh(core_axis_name="c", subcore_axis_name="s", num_cores=2, num_subcores=16)
    f = pl.kernel(
        _gather_body,
        out_type=jax.ShapeDtypeStruct((_P, _D), jnp.float32),
        mesh=mesh,
        compiler_params=pltpu.CompilerParams(needs_layout_passes=False),
        scratch_types=[
            pltpu.VMEM((_GR,), jnp.int32),
            pltpu.VMEM((_GR,), jnp.int32),
            pltpu.VMEM((_GR, _D), jnp.float32),
            pltpu.VMEM((_GR, _D), jnp.float32),
            pltpu.SemaphoreType.DMA,
            pltpu.SemaphoreType.DMA,
            pltpu.SemaphoreType.DMA,
            pltpu.SemaphoreType.DMA,
        ],
    )
    return f(x_flat, tok_slot)


def _group_body(ex_ref, xs_ref, g_ref, u_ref, d_ref, w_ref, y_ref):
    xb = xs_ref[...].astype(jnp.bfloat16)
    h1 = jnp.dot(xb, g_ref[0], preferred_element_type=jnp.float32)
    h2 = jnp.dot(xb, u_ref[0], preferred_element_type=jnp.float32)
    h = h1 * (1.0 / (1.0 + jnp.exp(-h1))) * h2 * w_ref[...]
    y_ref[...] = jnp.dot(h.astype(jnp.bfloat16), d_ref[0],
                         preferred_element_type=jnp.float32)


def _group_call(tile_ex, xs, gw16, uw16, dw16, w_slot):
    grid_spec = pltpu.PrefetchScalarGridSpec(
        num_scalar_prefetch=1,
        grid=(_G,),
        in_specs=[
            pl.BlockSpec((_TMG, _D), lambda i, ex: (i, 0)),
            pl.BlockSpec((1, _D, _F), lambda i, ex: (ex[i], 0, 0)),
            pl.BlockSpec((1, _D, _F), lambda i, ex: (ex[i], 0, 0)),
            pl.BlockSpec((1, _F, _D), lambda i, ex: (ex[i], 0, 0)),
            pl.BlockSpec((_TMG, 1), lambda i, ex: (i, 0)),
        ],
        out_specs=pl.BlockSpec((_TMG, _D), lambda i, ex: (i, 0)),
    )
    return pl.pallas_call(
        _group_body,
        grid_spec=grid_spec,
        out_shape=jax.ShapeDtypeStruct((_P, _D), jnp.float32),
        compiler_params=pltpu.CompilerParams(
            dimension_semantics=("arbitrary",)),
    )(tile_ex, xs, gw16, uw16, dw16, w_slot)


_CT = 8


def _combine_body(sh_hbm, y_hbm, inv_hbm, out_hbm,
                  idxa, idxb_, sh0, sh1, y0, y1, gs0, gs1, ws0, ws1):
    cid = lax.axis_index("c")
    sid = lax.axis_index("s")
    wid = sid * 2 + cid
    tbase = wid * (_N // _NW)
    nch = _N // _NW // _CT
    idxv = [idxa, idxb_]
    shb = [sh0, sh1]
    yb = [y0, y1]
    gsem = [gs0, gs1]
    wsem = [ws0, ws1]
    gdesc = [None, None]
    wdesc = [None, None]

    def stage(c, b):
        tb = tbase + c * _CT
        pltpu.sync_copy(inv_hbm.at[pl.ds(tb, _CT)], idxv[b].at[pl.ds(0, _CT)])
        pltpu.sync_copy(inv_hbm.at[pl.ds(_N + tb, _CT)],
                        idxv[b].at[pl.ds(_CT, _CT)])
        pltpu.sync_copy(sh_hbm.at[pl.ds(tb, _CT)], shb[b])
        return pltpu.async_copy(y_hbm.at[idxv[b]], yb[b], gsem[b])

    gdesc[0] = stage(0, 0)
    for c in range(nch):
        cb = c % 2
        nb = (c + 1) % 2
        if c + 1 < nch:
            if wdesc[nb] is not None:
                wdesc[nb].wait()
            gdesc[nb] = stage(c + 1, nb)
        gdesc[cb].wait()
        shv = shb[cb]
        yv = yb[cb]

        def add_rows(i, _, shv=shv, yv=yv):
            t = i // (_D // 16)
            cv = i % (_D // 16)
            s = (shv[t, pl.ds(cv * 16, 16)]
                 + yv[t, pl.ds(cv * 16, 16)]
                 + yv[t + _CT, pl.ds(cv * 16, 16)])
            shv[t, pl.ds(cv * 16, 16)] = s
            return 0

        lax.fori_loop(0, _CT * (_D // 16), add_rows, 0)
        wdesc[cb] = pltpu.async_copy(
            shv, out_hbm.at[pl.ds(tbase + c * _CT, _CT)], wsem[cb])
    for d in wdesc:
        if d is not None:
            d.wait()


def _combine_call(sh, y, inv):
    mesh = plsc.VectorSubcoreMesh(core_axis_name="c", subcore_axis_name="s", num_cores=2, num_subcores=16)
    f = pl.kernel(
        _combine_body,
        out_type=jax.ShapeDtypeStruct((_N, _D), jnp.float32),
        mesh=mesh,
        compiler_params=pltpu.CompilerParams(needs_layout_passes=False),
        scratch_types=[
            pltpu.VMEM((2 * _CT,), jnp.int32),
            pltpu.VMEM((2 * _CT,), jnp.int32),
            pltpu.VMEM((_CT, _D), jnp.float32),
            pltpu.VMEM((_CT, _D), jnp.float32),
            pltpu.VMEM((2 * _CT, _D), jnp.float32),
            pltpu.VMEM((2 * _CT, _D), jnp.float32),
            pltpu.SemaphoreType.DMA,
            pltpu.SemaphoreType.DMA,
            pltpu.SemaphoreType.DMA,
            pltpu.SemaphoreType.DMA,
        ],
    )
    return f(sh, y, inv)


@jax.jit
def kernel(x, router_w, shared_gate, shared_up, shared_down, gate_w, up_w, down_w):
    x_flat = x.reshape(_N, _D)
    sg16 = shared_gate.astype(jnp.bfloat16)
    su16 = shared_up.astype(jnp.bfloat16)
    sd16 = shared_down.astype(jnp.bfloat16)
    gw16 = gate_w.astype(jnp.bfloat16)
    uw16 = up_w.astype(jnp.bfloat16)
    dw16 = down_w.astype(jnp.bfloat16)

    eid2, val2, c1, c2 = _router_call(x_flat, router_w)
    eid_km = eid2.T.reshape(_NK)
    val_km = val2.T.reshape(_NK)
    counts = jnp.concatenate([c1.reshape(_N // 256, _E), c2.reshape(_N // 256, _E)], axis=0).reshape(_NW * _E)
    inv, tok_slot, w_slot, tile_ex = _dispatch_call(eid_km, val_km, counts)
    xs = _gather_call(x_flat, tok_slot)
    y = _group_call(tile_ex, xs, gw16, uw16, dw16, w_slot.reshape(_P, 1))
    sh = _shared_call(x_flat, sg16, su16, sd16)
    out = _combine_call(sh, y, inv)
    return out.reshape(_B, _T, _D)
